# baseline (speedup 1.0000x reference)
import functools

import jax
import jax.numpy as jnp
from jax import lax
from jax.experimental import pallas as pl
from jax.experimental.pallas import tpu as pltpu
from jax.experimental.pallas import tpu_sc as plsc

_N = 16384
_C = 512
_GROUP = 32
_SPLIT = 8
_DENOM = 6184.0

_NC = 2
_NS = 16
_NW = _NC * _NS

_N_SC = 4096
_ROWS_PER_W = _N_SC // _NW
_G_PER_W = _ROWS_PER_W // _GROUP
_WIN = 64
_NWIN = _ROWS_PER_W // _WIN
_NBUF = min(3, _NWIN)

_G_TC = (_N - _N_SC) // _GROUP
_G0_TC = _N_SC // _GROUP
_GBLK = 64


def _sc_body(pred_hbm, labels_hbm, out_hbm, lab_v, win_v, sel_v, part_v, sem):
    wid = lax.axis_index("s") * _NC + lax.axis_index("c")
    base = wid * _ROWS_PER_W

    lane = jax.lax.iota(jnp.int32, 16)

    def fire(w):
        return pltpu.async_copy(
            pred_hbm.at[pl.ds(base + w * _WIN, _WIN)], win_v.at[w % _NBUF], sem
        )

    copies = {w: fire(w) for w in range(_NBUF - 1)}
    pltpu.sync_copy(labels_hbm.at[pl.ds(base * 1, _ROWS_PER_W)], lab_v)

    for w in range(_NWIN):
        if w + _NBUF - 1 < _NWIN:
            copies[w + _NBUF - 1] = fire(w + _NBUF - 1)
        copies[w].wait()
        for v in range(_WIN // 16):
            ridx = w * _WIN + v * 16 + lane
            labs = plsc.load_gather(lab_v, [ridx])
            col = (labs + (_C - 1)) & (_C - 1)
            val = plsc.load_gather(win_v.at[w % _NBUF], [v * 16 + lane, col])
            sel_v[pl.ds(w * _WIN + v * 16, 16)] = val

    zero = jnp.zeros((16,), jnp.float32)
    one = jnp.full((16,), 1.0, jnp.float32)
    live = lane < _G_PER_W
    gclamp = jnp.minimum(lane, _G_PER_W - 1)
    acc = zero
    m0 = zero
    m1 = zero
    m2 = zero
    m3 = zero
    for j in range(_GROUP):
        v = plsc.load_gather(sel_v, [gclamp * _GROUP + j])
        if j < _SPLIT:
            acc = acc + jnp.where(live, jnp.maximum(zero, one - v), zero)
        else:
            x = jnp.where(live, jnp.maximum(zero, one + v), zero)
            t = jnp.maximum(m0, x)
            x = jnp.minimum(m0, x)
            m0 = t
            t = jnp.maximum(m1, x)
            x = jnp.minimum(m1, x)
            m1 = t
            t = jnp.maximum(m2, x)
            x = jnp.minimum(m2, x)
            m2 = t
            m3 = jnp.maximum(m3, x)
    part_v[...] = acc + ((m0 + m1) + (m2 + m3))
    pltpu.sync_copy(part_v, out_hbm.at[wid])


def _run_sc(pred2d, labels):
    mesh = plsc.VectorSubcoreMesh(core_axis_name="c", subcore_axis_name="s")
    run = pl.kernel(
        _sc_body,
        out_type=jax.ShapeDtypeStruct((_NW, 16), jnp.float32),
        mesh=mesh,
        scratch_types=[
            pltpu.VMEM((_ROWS_PER_W,), jnp.int32),
            pltpu.VMEM((_NBUF, _WIN, _C), jnp.float32),
            pltpu.VMEM((_ROWS_PER_W,), jnp.float32),
            pltpu.VMEM((16,), jnp.float32),
            pltpu.SemaphoreType.DMA,
        ],
        compiler_params=pltpu.CompilerParams(needs_layout_passes=False),
        name="ohem_completeness_loss_sc",
    )
    return run(pred2d, labels)


def _tc_body(lab_ref, x_ref, out_ref):
    pid = pl.program_id(0)
    x = x_ref[...]
    lab = lab_ref[...]
    col = (lab + (_C - 1)) & (_C - 1)
    cmat = lax.broadcasted_iota(jnp.int32, (_GBLK, _GROUP, _C), 2)
    neg_inf = jnp.float32(-3.0e38)
    sel = jnp.max(jnp.where(cmat == col[:, :, None], x, neg_inf), axis=2)
    j = lax.broadcasted_iota(jnp.int32, (_GBLK, _GROUP), 1)
    is_pos = j < _SPLIT
    sign = jnp.where(is_pos, 1.0, -1.0).astype(jnp.float32)
    loss = jnp.maximum(0.0, 1.0 - sign * sel)
    pos_sum = jnp.sum(jnp.where(is_pos, loss, 0.0))
    key = (lax.bitcast_convert_type(loss, jnp.int32) & ~31) | j
    key = jnp.where(is_pos, 0, key)
    neg_sum = jnp.float32(0.0)
    for _ in range(4):
        m = jnp.max(key, axis=1, keepdims=True)
        neg_sum = neg_sum + jnp.sum(
            lax.bitcast_convert_type(m & ~31, jnp.float32))
        key = jnp.where(key == m, 0, key)

    @pl.when(pid == 0)
    def _():
        out_ref[...] = jnp.zeros_like(out_ref)

    out_ref[...] += (pos_sum + neg_sum).reshape(1, 1)


def _run_tc(pred3, labels2):
    grid = (_G_TC // _GBLK,)
    return pl.pallas_call(
        _tc_body,
        grid=grid,
        in_specs=[
            pl.BlockSpec((_GBLK, _GROUP), lambda i: (_G0_TC // _GBLK + i, 0)),
            pl.BlockSpec((_GBLK, _GROUP, _C),
                         lambda i: (_G0_TC // _GBLK + i, 0, 0)),
        ],
        out_specs=pl.BlockSpec((1, 1), lambda i: (0, 0)),
        out_shape=jax.ShapeDtypeStruct((1, 1), jnp.float32),
        compiler_params=pltpu.CompilerParams(
            dimension_semantics=("arbitrary",)),
        name="ohem_completeness_loss_tc",
    )(labels2, pred3)


@jax.jit
def _ohem(pred, labels):
    sc_parts = _run_sc(pred, labels)
    pred3 = pred.reshape(_N // _GROUP, _GROUP, _C)
    labels2 = labels.reshape(_N // _GROUP, _GROUP)
    tc_part = _run_tc(pred3, labels2)
    return (jnp.sum(sc_parts) + tc_part[0, 0]) * (1.0 / _DENOM)


def kernel(pred, labels, sample_split, sample_group_size):
    loss = _ohem(pred, labels)
    loss = loss + 0.0 * (sample_split + sample_group_size)
    return loss.reshape(1)


# speedup vs baseline: 1.0791x; 1.0791x over previous
import jax
import jax.numpy as jnp
from jax import lax
from jax.experimental import pallas as pl
from jax.experimental.pallas import tpu as pltpu
from jax.experimental.pallas import tpu_sc as plsc

_N = 16384
_C = 512
_GROUP = 32
_SPLIT = 8
_DENOM = 6184.0

_NC = 2
_NS = 16
_NW = _NC * _NS
_ROWS_PER_W = _N // _NW
_WIN = 32
_NWIN = _ROWS_PER_W // _WIN
_NBUF = 6


def _sc_body(pred_hbm, labels_hbm, out_hbm, lab_v, win_v, sel_v, part_v, sem):
    wid = lax.axis_index("s") * _NC + lax.axis_index("c")
    base = wid * _ROWS_PER_W

    lane = jax.lax.iota(jnp.int32, 16)

    def fire(w):
        return pltpu.async_copy(
            pred_hbm.at[pl.ds(base + w * _WIN, _WIN)], win_v.at[w % _NBUF], sem
        )

    copies = {w: fire(w) for w in range(_NBUF - 1)}
    pltpu.sync_copy(labels_hbm.at[pl.ds(base * 1, _ROWS_PER_W)], lab_v)

    for w in range(_NWIN):
        if w + _NBUF - 1 < _NWIN:
            copies[w + _NBUF - 1] = fire(w + _NBUF - 1)
        copies[w].wait()
        for v in range(_WIN // 16):
            ridx = w * _WIN + v * 16 + lane
            labs = plsc.load_gather(lab_v, [ridx])
            col = (labs + (_C - 1)) & (_C - 1)
            val = plsc.load_gather(win_v.at[w % _NBUF], [v * 16 + lane, col])
            sel_v[pl.ds(w * _WIN + v * 16, 16)] = val

    zero = jnp.zeros((16,), jnp.float32)
    one = jnp.full((16,), 1.0, jnp.float32)
    acc = zero
    m0 = zero
    m1 = zero
    m2 = zero
    m3 = zero
    for j in range(_GROUP):
        v = plsc.load_gather(sel_v, [lane * _GROUP + j])
        if j < _SPLIT:
            acc = acc + jnp.maximum(zero, one - v)
        else:
            x = jnp.maximum(zero, one + v)
            t = jnp.maximum(m0, x)
            x = jnp.minimum(m0, x)
            m0 = t
            t = jnp.maximum(m1, x)
            x = jnp.minimum(m1, x)
            m1 = t
            t = jnp.maximum(m2, x)
            x = jnp.minimum(m2, x)
            m2 = t
            m3 = jnp.maximum(m3, x)
    part_v[...] = acc + ((m0 + m1) + (m2 + m3))
    pltpu.sync_copy(part_v, out_hbm.at[wid])


@jax.jit
def _ohem_sc(pred2d, labels):
    mesh = plsc.VectorSubcoreMesh(core_axis_name="c", subcore_axis_name="s")
    run = pl.kernel(
        _sc_body,
        out_type=jax.ShapeDtypeStruct((_NW, 16), jnp.float32),
        mesh=mesh,
        scratch_types=[
            pltpu.VMEM((_ROWS_PER_W,), jnp.int32),
            pltpu.VMEM((_NBUF, _WIN, _C), jnp.float32),
            pltpu.VMEM((_ROWS_PER_W,), jnp.float32),
            pltpu.VMEM((16,), jnp.float32),
            pltpu.SemaphoreType.DMA,
        ],
        compiler_params=pltpu.CompilerParams(needs_layout_passes=False),
        name="ohem_completeness_loss",
    )
    return run(pred2d, labels)


def kernel(pred, labels, sample_split, sample_group_size):
    parts = _ohem_sc(pred, labels)
    loss = jnp.sum(parts) * (1.0 / _DENOM)
    loss = loss + 0.0 * (sample_split + sample_group_size)
    return loss.reshape(1)


# speedup vs baseline: 1.0841x; 1.0047x over previous
import jax
import jax.numpy as jnp
from jax import lax
from jax.experimental import pallas as pl
from jax.experimental.pallas import tpu as pltpu
from jax.experimental.pallas import tpu_sc as plsc

_N = 16384
_C = 512
_GROUP = 32
_SPLIT = 8
_DENOM = 6184.0

_NC = 2
_NS = 16
_NW = _NC * _NS
_ROWS_PER_W = _N // _NW
_WIN = 64
_NWIN = _ROWS_PER_W // _WIN
_NBUF = 3


def _sc_body(pred_hbm, labels_hbm, out_hbm, lab_v, win_v, sel_v, part_v, sem):
    wid = lax.axis_index("s") * _NC + lax.axis_index("c")
    base = wid * _ROWS_PER_W

    lane = jax.lax.iota(jnp.int32, 16)

    def fire(w):
        return pltpu.async_copy(
            pred_hbm.at[pl.ds(base + w * _WIN, _WIN)], win_v.at[w % _NBUF], sem
        )

    copies = {w: fire(w) for w in range(_NBUF - 1)}
    pltpu.sync_copy(labels_hbm.at[pl.ds(base * 1, _ROWS_PER_W)], lab_v)

    for w in range(_NWIN):
        if w + _NBUF - 1 < _NWIN:
            copies[w + _NBUF - 1] = fire(w + _NBUF - 1)
        copies[w].wait()
        for v in range(_WIN // 16):
            ridx = w * _WIN + v * 16 + lane
            labs = plsc.load_gather(lab_v, [ridx])
            col = (labs + (_C - 1)) & (_C - 1)
            val = plsc.load_gather(win_v.at[w % _NBUF], [v * 16 + lane, col])
            sel_v[pl.ds(w * _WIN + v * 16, 16)] = val

    zero = jnp.zeros((16,), jnp.float32)
    one = jnp.full((16,), 1.0, jnp.float32)
    acc = zero
    m0 = zero
    m1 = zero
    m2 = zero
    m3 = zero
    for j in range(_GROUP):
        v = plsc.load_gather(sel_v, [lane * _GROUP + j])
        if j < _SPLIT:
            acc = acc + jnp.maximum(zero, one - v)
        else:
            x = jnp.maximum(zero, one + v)
            t = jnp.maximum(m0, x)
            x = jnp.minimum(m0, x)
            m0 = t
            t = jnp.maximum(m1, x)
            x = jnp.minimum(m1, x)
            m1 = t
            t = jnp.maximum(m2, x)
            x = jnp.minimum(m2, x)
            m2 = t
            m3 = jnp.maximum(m3, x)
    part_v[...] = acc + ((m0 + m1) + (m2 + m3))
    pltpu.sync_copy(part_v, out_hbm.at[wid])


@jax.jit
def _ohem_sc(pred2d, labels):
    mesh = plsc.VectorSubcoreMesh(core_axis_name="c", subcore_axis_name="s")
    run = pl.kernel(
        _sc_body,
        out_type=jax.ShapeDtypeStruct((_NW, 16), jnp.float32),
        mesh=mesh,
        scratch_types=[
            pltpu.VMEM((_ROWS_PER_W,), jnp.int32),
            pltpu.VMEM((_NBUF, _WIN, _C), jnp.float32),
            pltpu.VMEM((_ROWS_PER_W,), jnp.float32),
            pltpu.VMEM((16,), jnp.float32),
            pltpu.SemaphoreType.DMA,
        ],
        compiler_params=pltpu.CompilerParams(needs_layout_passes=False),
        name="ohem_completeness_loss",
    )
    return run(pred2d, labels)


def kernel(pred, labels, sample_split, sample_group_size):
    parts = _ohem_sc(pred, labels)
    loss = jnp.sum(parts) * (1.0 / _DENOM)
    loss = loss + 0.0 * (sample_split + sample_group_size)
    return loss.reshape(1)
